# baseline (speedup 1.0000x reference)
import functools

import jax
import jax.numpy as jnp
from jax import lax
from jax.experimental import pallas as pl
from jax.experimental.pallas import tpu as pltpu
from jax.experimental.pallas import tpu_sc as plsc

VOCAB = 100000
HIDDEN = 64
RANK = 16
BATCH = 4096
SEQ = 200
N = BATCH * SEQ

VBLK = 1024
NBLK = -(-VOCAB // VBLK)
TROWS = NBLK * (VBLK // 2)

NC = 2
NS = 16
NW = NC * NS
RPW = BATCH // NW
CHA = 128
CHB = SEQ - CHA
NBUF = 4
NGROUP = RPW // NBUF


def _fuse_body(at_ref, b_ref, baset_ref, out_ref):
    delta_t = lax.dot_general(
        b_ref[...], at_ref[...], (((0,), (0,)), ((), ())),
        preferred_element_type=jnp.float32,
    )
    res = jnp.transpose(baset_ref[...] + delta_t)
    out_ref[:, 0:64] = res[0 : VBLK // 2]
    out_ref[:, 64:128] = res[VBLK // 2 : VBLK]


def _fused_table(base_t, lora_a_t, lora_B):
    return pl.pallas_call(
        _fuse_body,
        grid=(NBLK,),
        in_specs=[
            pl.BlockSpec((RANK, VBLK), lambda i: (0, i)),
            pl.BlockSpec((RANK, HIDDEN), lambda i: (0, 0)),
            pl.BlockSpec((HIDDEN, VBLK), lambda i: (0, i)),
        ],
        out_specs=pl.BlockSpec((VBLK // 2, 128), lambda i: (i, 0)),
        out_shape=jax.ShapeDtypeStruct((TROWS, 128), jnp.float32),
    )(lora_a_t, lora_B, base_t)


_RBLK = 512


def _remap_body(idst_ref, outa_ref, outb_ref):
    v = idst_ref[...]
    u = (v & (-1024)) | ((v & 511) << 1) | ((v >> 9) & 1)
    ut = jnp.transpose(u)
    outa_ref[...] = ut[:, 0:CHA]
    outb_ref[...] = jnp.concatenate(
        [ut[:, CHA:SEQ], jnp.zeros((_RBLK, CHA - CHB), jnp.int32)], axis=1
    )


def _remap_ids(ids_t):
    return pl.pallas_call(
        _remap_body,
        grid=(BATCH // _RBLK,),
        in_specs=[pl.BlockSpec((SEQ, _RBLK), lambda i: (0, i))],
        out_specs=[
            pl.BlockSpec((_RBLK, CHA), lambda i: (i, 0)),
            pl.BlockSpec((_RBLK, CHA), lambda i: (i, 0)),
        ],
        out_shape=[
            jax.ShapeDtypeStruct((BATCH, CHA), jnp.int32),
            jax.ShapeDtypeStruct((BATCH, CHA), jnp.int32),
        ],
    )(ids_t)


_mesh = plsc.VectorSubcoreMesh(
    core_axis_name="c", subcore_axis_name="s", num_cores=NC, num_subcores=NS
)


@functools.partial(
    pl.kernel,
    out_type=jax.ShapeDtypeStruct((BATCH, SEQ, HIDDEN), jnp.float32),
    mesh=_mesh,
    scratch_types=[
        pltpu.VMEM((RPW, CHA), jnp.int32),
        pltpu.VMEM((RPW, CHA), jnp.int32),
        [pltpu.VMEM((CHA, HIDDEN), jnp.float32) for _ in range(NBUF)],
        [pltpu.VMEM((CHB, HIDDEN), jnp.float32) for _ in range(NBUF)],
        [pltpu.SemaphoreType.DMA for _ in range(NBUF)],
        [pltpu.SemaphoreType.DMA for _ in range(NBUF)],
        [pltpu.SemaphoreType.DMA for _ in range(NBUF)],
        [pltpu.SemaphoreType.DMA for _ in range(NBUF)],
    ],
    compiler_params=pltpu.CompilerParams(use_tc_tiling_on_sc=False),
)
def _sc_gather(table_hbm, idxa_hbm, idxb_hbm, out_hbm,
               idxa_v, idxb_v, bufA, bufB, gsA, gsB, osA, osB):
    wid = lax.axis_index("s") * NC + lax.axis_index("c")
    row0 = wid * RPW
    pltpu.sync_copy(idxa_hbm.at[pl.ds(row0, RPW)], idxa_v)
    pltpu.sync_copy(idxb_hbm.at[pl.ds(row0, RPW)], idxb_v)

    def start_gather(r, b):
        pltpu.async_copy(table_hbm.at[idxa_v.at[r]], bufA[b], gsA[b])
        pltpu.async_copy(table_hbm.at[idxb_v.at[r, pl.ds(0, CHB)]], bufB[b], gsB[b])

    def wait_gather(b):
        pltpu.make_async_copy(table_hbm.at[idxa_v.at[0]], bufA[b], gsA[b]).wait()
        pltpu.make_async_copy(
            table_hbm.at[idxb_v.at[0, pl.ds(0, CHB)]], bufB[b], gsB[b]
        ).wait()

    def start_out(r, b):
        row = row0 + r
        pltpu.async_copy(bufA[b], out_hbm.at[row, pl.ds(0, CHA)], osA[b])
        pltpu.async_copy(bufB[b], out_hbm.at[row, pl.ds(CHA, CHB)], osB[b])

    def wait_out(b):
        pltpu.make_async_copy(bufA[b], out_hbm.at[0, pl.ds(0, CHA)], osA[b]).wait()
        pltpu.make_async_copy(bufB[b], out_hbm.at[0, pl.ds(CHA, CHB)], osB[b]).wait()

    for b in range(NBUF):
        start_gather(b, b)

    def group(g, carry):
        for b in range(NBUF):
            wait_gather(b)
            start_out(g * NBUF + b, b)
        for b in range(NBUF):
            nr = (g + 1) * NBUF + b

            @pl.when(nr < RPW)
            def _():
                wait_out(b)
                start_gather(nr, b)

        return carry

    lax.fori_loop(0, NGROUP, group, 0)
    for b in range(NBUF):
        wait_out(b)


_TSB = 512
_TSS = 2
_TGRID = (SEQ // _TSS) * (BATCH // _TSB)


def _otr_body(x_hbm, out_ref, buf, sems):
    g = pl.program_id(0)
    slot = lax.rem(g, 2)

    def start(step, sl):
        si = step // (BATCH // _TSB)
        bi = lax.rem(step, BATCH // _TSB)
        pltpu.make_async_copy(
            x_hbm.at[pl.ds(bi * _TSB, _TSB), pl.ds(si * _TSS, _TSS), :],
            buf.at[sl],
            sems.at[sl],
        ).start()

    @pl.when(g == 0)
    def _():
        start(g, slot)

    @pl.when(g + 1 < _TGRID)
    def _():
        start(g + 1, 1 - slot)

    pltpu.make_async_copy(
        x_hbm.at[pl.ds(0, _TSB), pl.ds(0, _TSS), :], buf.at[slot], sems.at[slot]
    ).wait()
    w = buf[slot]
    for k in range(_TSS):
        out_ref[k, :, :] = jnp.transpose(w[:, k, :])


def _out_transpose(x):
    return pl.pallas_call(
        _otr_body,
        grid=(_TGRID,),
        in_specs=[pl.BlockSpec(memory_space=pl.ANY)],
        out_specs=pl.BlockSpec(
            (_TSS, HIDDEN, _TSB),
            lambda g: (g // (BATCH // _TSB), 0, g % (BATCH // _TSB)),
        ),
        out_shape=jax.ShapeDtypeStruct((SEQ, HIDDEN, BATCH), jnp.float32),
        scratch_shapes=[
            pltpu.VMEM((2, _TSB, _TSS, HIDDEN), jnp.float32),
            pltpu.SemaphoreType.DMA((2,)),
        ],
    )(x)


def kernel(input_ids, base_table, lora_A, lora_B):
    tab = _fused_table(base_table.T, lora_A.T, lora_B)
    ua, ub = _remap_ids(input_ids.astype(jnp.int32).T)
    out = _sc_gather(tab.reshape(2 * TROWS, HIDDEN), ua, ub)
    p = _out_transpose(out)
    return jnp.transpose(p, (2, 0, 1))


# speedup vs baseline: 1.4065x; 1.4065x over previous
import functools

import jax
import jax.numpy as jnp
from jax import lax
from jax.experimental import pallas as pl
from jax.experimental.pallas import tpu as pltpu
from jax.experimental.pallas import tpu_sc as plsc

VOCAB = 100000
HIDDEN = 64
RANK = 16
BATCH = 4096
SEQ = 200
N = BATCH * SEQ

VBLK = 1024
NBLK = -(-VOCAB // VBLK)
TROWS = NBLK * (VBLK // 2)

NC = 2
NS = 16
NW = NC * NS
RPW = BATCH // NW
CHA = 128
CHB = SEQ - CHA
NBUF = 4
NGROUP = RPW // NBUF


def _fuse_body(at_ref, b_ref, baset_ref, out_ref):
    delta_t = lax.dot_general(
        b_ref[...], at_ref[...], (((0,), (0,)), ((), ())),
        preferred_element_type=jnp.float32,
    )
    res = jnp.transpose(baset_ref[...] + delta_t)
    out_ref[:, 0:64] = res[0 : VBLK // 2]
    out_ref[:, 64:128] = res[VBLK // 2 : VBLK]


def _fused_table(base_t, lora_a_t, lora_B):
    return pl.pallas_call(
        _fuse_body,
        grid=(NBLK,),
        in_specs=[
            pl.BlockSpec((RANK, VBLK), lambda i: (0, i)),
            pl.BlockSpec((RANK, HIDDEN), lambda i: (0, 0)),
            pl.BlockSpec((HIDDEN, VBLK), lambda i: (0, i)),
        ],
        out_specs=pl.BlockSpec((VBLK // 2, 128), lambda i: (i, 0)),
        out_shape=jax.ShapeDtypeStruct((TROWS, 128), jnp.float32),
    )(lora_a_t, lora_B, base_t)


_RBLK = 512


def _remap_body(idst_ref, outa_ref, outb_ref):
    v = idst_ref[...]
    u = (v & (-1024)) | ((v & 511) << 1) | ((v >> 9) & 1)
    ut = jnp.transpose(u)
    outa_ref[...] = ut[:, 0:CHA]
    outb_ref[...] = jnp.concatenate(
        [ut[:, CHA:SEQ], jnp.zeros((_RBLK, CHA - CHB), jnp.int32)], axis=1
    )


def _remap_ids(ids_t):
    return pl.pallas_call(
        _remap_body,
        grid=(BATCH // _RBLK,),
        in_specs=[pl.BlockSpec((SEQ, _RBLK), lambda i: (0, i))],
        out_specs=[
            pl.BlockSpec((_RBLK, CHA), lambda i: (i, 0)),
            pl.BlockSpec((_RBLK, CHA), lambda i: (i, 0)),
        ],
        out_shape=[
            jax.ShapeDtypeStruct((BATCH, CHA), jnp.int32),
            jax.ShapeDtypeStruct((BATCH, CHA), jnp.int32),
        ],
    )(ids_t)


_mesh = plsc.VectorSubcoreMesh(
    core_axis_name="c", subcore_axis_name="s", num_cores=NC, num_subcores=NS
)


@functools.partial(
    pl.kernel,
    out_type=jax.ShapeDtypeStruct((BATCH, SEQ, HIDDEN), jnp.float32),
    mesh=_mesh,
    scratch_types=[
        pltpu.VMEM((RPW, CHA), jnp.int32),
        pltpu.VMEM((RPW, CHA), jnp.int32),
        [pltpu.VMEM((CHA, HIDDEN), jnp.float32) for _ in range(NBUF)],
        [pltpu.VMEM((CHB, HIDDEN), jnp.float32) for _ in range(NBUF)],
        [pltpu.SemaphoreType.DMA for _ in range(NBUF)],
        [pltpu.SemaphoreType.DMA for _ in range(NBUF)],
        [pltpu.SemaphoreType.DMA for _ in range(NBUF)],
        [pltpu.SemaphoreType.DMA for _ in range(NBUF)],
    ],
    compiler_params=pltpu.CompilerParams(use_tc_tiling_on_sc=False),
)
def _sc_gather(table_hbm, idxa_hbm, idxb_hbm, out_hbm,
               idxa_v, idxb_v, bufA, bufB, gsA, gsB, osA, osB):
    wid = lax.axis_index("s") * NC + lax.axis_index("c")
    row0 = wid * RPW
    pltpu.sync_copy(idxa_hbm.at[pl.ds(row0, RPW)], idxa_v)
    pltpu.sync_copy(idxb_hbm.at[pl.ds(row0, RPW)], idxb_v)

    def start_gather(r, b):
        pltpu.async_copy(table_hbm.at[idxa_v.at[r]], bufA[b], gsA[b])
        pltpu.async_copy(table_hbm.at[idxb_v.at[r, pl.ds(0, CHB)]], bufB[b], gsB[b])

    def wait_gather(b):
        pltpu.make_async_copy(table_hbm.at[idxa_v.at[0]], bufA[b], gsA[b]).wait()
        pltpu.make_async_copy(
            table_hbm.at[idxb_v.at[0, pl.ds(0, CHB)]], bufB[b], gsB[b]
        ).wait()

    def start_out(r, b):
        row = row0 + r
        pltpu.async_copy(bufA[b], out_hbm.at[row, pl.ds(0, CHA)], osA[b])
        pltpu.async_copy(bufB[b], out_hbm.at[row, pl.ds(CHA, CHB)], osB[b])

    def wait_out(b):
        pltpu.make_async_copy(bufA[b], out_hbm.at[0, pl.ds(0, CHA)], osA[b]).wait()
        pltpu.make_async_copy(bufB[b], out_hbm.at[0, pl.ds(CHA, CHB)], osB[b]).wait()

    for b in range(NBUF):
        start_gather(b, b)

    def group(g, carry):
        for b in range(NBUF):
            wait_gather(b)
            start_out(g * NBUF + b, b)
        for b in range(NBUF):
            nr = (g + 1) * NBUF + b

            @pl.when(nr < RPW)
            def _():
                wait_out(b)
                start_gather(nr, b)

        return carry

    lax.fori_loop(0, NGROUP, group, 0)
    for b in range(NBUF):
        wait_out(b)


_TSB = 512
_TSS = 8
_NSB = BATCH // _TSB
_NSS = SEQ // _TSS
_TGRID = _NSS * _NSB


def _otr_body(x_hbm, out_ref, buf, sems):
    si = pl.program_id(0)
    bi = pl.program_id(1)
    g = si * _NSB + bi
    slot = lax.rem(g, 2)

    def start(step, sl):
        s = step // _NSB
        b = lax.rem(step, _NSB)
        pltpu.make_async_copy(
            x_hbm.at[pl.ds(b * _TSB, _TSB), pl.ds(s * _TSS, _TSS), :],
            buf.at[sl],
            sems.at[sl],
        ).start()

    @pl.when(g == 0)
    def _():
        start(g, slot)

    @pl.when(g + 1 < _TGRID)
    def _():
        start(g + 1, 1 - slot)

    pltpu.make_async_copy(
        x_hbm.at[pl.ds(0, _TSB), pl.ds(0, _TSS), :], buf.at[slot], sems.at[slot]
    ).wait()
    w = buf[slot].reshape(_TSB, _TSS * HIDDEN)
    for k in range(_TSS * HIDDEN // 128):
        r = jnp.transpose(w[:, k * 128 : (k + 1) * 128])
        out_ref[pl.ds(2 * k, 2)] = r.reshape(2, HIDDEN, _TSB)


def _out_transpose(x):
    return pl.pallas_call(
        _otr_body,
        grid=(_NSS, _NSB),
        in_specs=[pl.BlockSpec(memory_space=pl.ANY)],
        out_specs=pl.BlockSpec(
            (_TSS, HIDDEN, _TSB), lambda si, bi: (si, 0, bi)
        ),
        out_shape=jax.ShapeDtypeStruct((SEQ, HIDDEN, BATCH), jnp.float32),
        scratch_shapes=[
            pltpu.VMEM((2, _TSB, _TSS, HIDDEN), jnp.float32),
            pltpu.SemaphoreType.DMA((2,)),
        ],
    )(x)


def kernel(input_ids, base_table, lora_A, lora_B):
    tab = _fused_table(base_table.T, lora_A.T, lora_B)
    ua, ub = _remap_ids(input_ids.astype(jnp.int32).T)
    out = _sc_gather(tab.reshape(2 * TROWS, HIDDEN), ua, ub)
    p = _out_transpose(out)
    return jnp.transpose(p, (2, 0, 1))


# speedup vs baseline: 1.5542x; 1.1050x over previous
import functools

import jax
import jax.numpy as jnp
from jax import lax
from jax.experimental import pallas as pl
from jax.experimental.pallas import tpu as pltpu
from jax.experimental.pallas import tpu_sc as plsc

VOCAB = 100000
HIDDEN = 64
RANK = 16
BATCH = 4096
SEQ = 200
N = BATCH * SEQ

VBLK = 1024
NBLK = -(-VOCAB // VBLK)
TROWS = NBLK * (VBLK // 2)

NC = 2
NS = 16
NW = NC * NS
RPW = BATCH // NW
CHA = 128
CHB = SEQ - CHA
NBUF = 4
NGROUP = RPW // NBUF


def _fuse_body(at_ref, b_ref, baset_ref, out_ref):
    delta_t = lax.dot_general(
        b_ref[...], at_ref[...], (((0,), (0,)), ((), ())),
        preferred_element_type=jnp.float32,
    )
    res = jnp.transpose(baset_ref[...] + delta_t)
    out_ref[:, 0:64] = res[0 : VBLK // 2]
    out_ref[:, 64:128] = res[VBLK // 2 : VBLK]


def _fused_table(base_t, lora_a_t, lora_B):
    return pl.pallas_call(
        _fuse_body,
        grid=(NBLK,),
        in_specs=[
            pl.BlockSpec((RANK, VBLK), lambda i: (0, i)),
            pl.BlockSpec((RANK, HIDDEN), lambda i: (0, 0)),
            pl.BlockSpec((HIDDEN, VBLK), lambda i: (0, i)),
        ],
        out_specs=pl.BlockSpec((VBLK // 2, 128), lambda i: (i, 0)),
        out_shape=jax.ShapeDtypeStruct((TROWS, 128), jnp.float32),
    )(lora_a_t, lora_B, base_t)


_RBLK = 512


def _remap_body(idst_ref, outa_ref, outb_ref):
    v = idst_ref[...]
    u = (v & (-1024)) | ((v & 511) << 1) | ((v >> 9) & 1)
    ut = jnp.transpose(u)
    outa_ref[...] = ut[:, 0:CHA]
    outb_ref[...] = jnp.concatenate(
        [ut[:, CHA:SEQ], jnp.zeros((_RBLK, CHA - CHB), jnp.int32)], axis=1
    )


def _remap_ids(ids_t):
    return pl.pallas_call(
        _remap_body,
        grid=(BATCH // _RBLK,),
        in_specs=[pl.BlockSpec((SEQ, _RBLK), lambda i: (0, i))],
        out_specs=[
            pl.BlockSpec((_RBLK, CHA), lambda i: (i, 0)),
            pl.BlockSpec((_RBLK, CHA), lambda i: (i, 0)),
        ],
        out_shape=[
            jax.ShapeDtypeStruct((BATCH, CHA), jnp.int32),
            jax.ShapeDtypeStruct((BATCH, CHA), jnp.int32),
        ],
    )(ids_t)


_mesh = plsc.VectorSubcoreMesh(
    core_axis_name="c", subcore_axis_name="s", num_cores=NC, num_subcores=NS
)


@functools.partial(
    pl.kernel,
    out_type=jax.ShapeDtypeStruct((BATCH, SEQ, HIDDEN), jnp.float32),
    mesh=_mesh,
    scratch_types=[
        pltpu.VMEM((RPW, CHA), jnp.int32),
        pltpu.VMEM((RPW, CHA), jnp.int32),
        [pltpu.VMEM((CHA, HIDDEN), jnp.float32) for _ in range(NBUF)],
        [pltpu.VMEM((CHB, HIDDEN), jnp.float32) for _ in range(NBUF)],
        [pltpu.SemaphoreType.DMA for _ in range(NBUF)],
        [pltpu.SemaphoreType.DMA for _ in range(NBUF)],
        [pltpu.SemaphoreType.DMA for _ in range(NBUF)],
        [pltpu.SemaphoreType.DMA for _ in range(NBUF)],
    ],
    compiler_params=pltpu.CompilerParams(use_tc_tiling_on_sc=False),
)
def _sc_gather(table_hbm, idxa_hbm, idxb_hbm, out_hbm,
               idxa_v, idxb_v, bufA, bufB, gsA, gsB, osA, osB):
    wid = lax.axis_index("s") * NC + lax.axis_index("c")
    row0 = wid * RPW
    pltpu.sync_copy(idxa_hbm.at[pl.ds(row0, RPW)], idxa_v)
    pltpu.sync_copy(idxb_hbm.at[pl.ds(row0, RPW)], idxb_v)

    def start_gather(r, b):
        pltpu.async_copy(table_hbm.at[idxa_v.at[r]], bufA[b], gsA[b])
        pltpu.async_copy(table_hbm.at[idxb_v.at[r, pl.ds(0, CHB)]], bufB[b], gsB[b])

    def wait_gather(b):
        pltpu.make_async_copy(table_hbm.at[idxa_v.at[0]], bufA[b], gsA[b]).wait()
        pltpu.make_async_copy(
            table_hbm.at[idxb_v.at[0, pl.ds(0, CHB)]], bufB[b], gsB[b]
        ).wait()

    def start_out(r, b):
        row = row0 + r
        pltpu.async_copy(bufA[b], out_hbm.at[row, pl.ds(0, CHA)], osA[b])
        pltpu.async_copy(bufB[b], out_hbm.at[row, pl.ds(CHA, CHB)], osB[b])

    def wait_out(b):
        pltpu.make_async_copy(bufA[b], out_hbm.at[0, pl.ds(0, CHA)], osA[b]).wait()
        pltpu.make_async_copy(bufB[b], out_hbm.at[0, pl.ds(CHA, CHB)], osB[b]).wait()

    for b in range(NBUF):
        start_gather(b, b)

    def group(g, carry):
        for b in range(NBUF):
            wait_gather(b)
            start_out(g * NBUF + b, b)
        for b in range(NBUF):
            nr = (g + 1) * NBUF + b

            @pl.when(nr < RPW)
            def _():
                wait_out(b)
                start_gather(nr, b)

        return carry

    lax.fori_loop(0, NGROUP, group, 0)
    for b in range(NBUF):
        wait_out(b)


_TB = 128
_TSTEPS = BATCH // _TB


def _otr_body(x_hbm, out_ref, buf, sems):
    g = pl.program_id(0)
    slot = lax.rem(g, 2)

    def start(step, sl):
        pltpu.make_async_copy(
            x_hbm.at[pl.ds(step * _TB, _TB)], buf.at[sl], sems.at[sl]
        ).start()

    @pl.when(g == 0)
    def _():
        start(g, slot)

    @pl.when(g + 1 < _TSTEPS)
    def _():
        start(g + 1, 1 - slot)

    pltpu.make_async_copy(
        x_hbm.at[pl.ds(0, _TB)], buf.at[slot], sems.at[slot]
    ).wait()
    w = buf[slot]
    for k in range(SEQ // 2):
        r = jnp.transpose(w[:, 2 * k : 2 * k + 2, :].reshape(_TB, 128))
        out_ref[pl.ds(2 * k, 2)] = r.reshape(2, HIDDEN, _TB)


def _out_transpose(x):
    return pl.pallas_call(
        _otr_body,
        grid=(_TSTEPS,),
        in_specs=[pl.BlockSpec(memory_space=pl.ANY)],
        out_specs=pl.BlockSpec((SEQ, HIDDEN, _TB), lambda g: (0, 0, g)),
        out_shape=jax.ShapeDtypeStruct((SEQ, HIDDEN, BATCH), jnp.float32),
        scratch_shapes=[
            pltpu.VMEM((2, _TB, SEQ, HIDDEN), jnp.float32),
            pltpu.SemaphoreType.DMA((2,)),
        ],
    )(x)


def kernel(input_ids, base_table, lora_A, lora_B):
    tab = _fused_table(base_table.T, lora_A.T, lora_B)
    ua, ub = _remap_ids(input_ids.astype(jnp.int32).T)
    out = _sc_gather(tab.reshape(2 * TROWS, HIDDEN), ua, ub)
    p = _out_transpose(out)
    return jnp.transpose(p, (2, 0, 1))


# speedup vs baseline: 1.5886x; 1.0221x over previous
import functools

import jax
import jax.numpy as jnp
from jax import lax
from jax.experimental import pallas as pl
from jax.experimental.pallas import tpu as pltpu
from jax.experimental.pallas import tpu_sc as plsc

VOCAB = 100000
HIDDEN = 64
RANK = 16
BATCH = 4096
SEQ = 200
N = BATCH * SEQ

VBLK = 1024
NBLK = -(-VOCAB // VBLK)
TROWS = NBLK * (VBLK // 2)

NC = 2
NS = 16
NW = NC * NS
NHALF = 2
HB = BATCH // NHALF
RPW = HB // NW
CHA = 128
CHB = SEQ - CHA
NBUF = 4
NGROUP = RPW // NBUF


def _fuse_body(at_ref, b_ref, baset_ref, out_ref):
    delta_t = lax.dot_general(
        b_ref[...], at_ref[...], (((0,), (0,)), ((), ())),
        preferred_element_type=jnp.float32,
    )
    res = jnp.transpose(baset_ref[...] + delta_t)
    out_ref[:, 0:64] = res[0 : VBLK // 2]
    out_ref[:, 64:128] = res[VBLK // 2 : VBLK]


def _fused_table(base_t, lora_a_t, lora_B):
    return pl.pallas_call(
        _fuse_body,
        grid=(NBLK,),
        in_specs=[
            pl.BlockSpec((RANK, VBLK), lambda i: (0, i)),
            pl.BlockSpec((RANK, HIDDEN), lambda i: (0, 0)),
            pl.BlockSpec((HIDDEN, VBLK), lambda i: (0, i)),
        ],
        out_specs=pl.BlockSpec((VBLK // 2, 128), lambda i: (i, 0)),
        out_shape=jax.ShapeDtypeStruct((TROWS, 128), jnp.float32),
    )(lora_a_t, lora_B, base_t)


_RBLK = 512


def _remap_body(idst_ref, outa_ref, outb_ref):
    v = idst_ref[...]
    u = (v & (-1024)) | ((v & 511) << 1) | ((v >> 9) & 1)
    ut = jnp.transpose(u)
    outa_ref[...] = ut[:, 0:CHA]
    outb_ref[...] = jnp.concatenate(
        [ut[:, CHA:SEQ], jnp.zeros((_RBLK, CHA - CHB), jnp.int32)], axis=1
    )


def _remap_ids(ids_t):
    return pl.pallas_call(
        _remap_body,
        grid=(BATCH // _RBLK,),
        in_specs=[pl.BlockSpec((SEQ, _RBLK), lambda i: (0, i))],
        out_specs=[
            pl.BlockSpec((_RBLK, CHA), lambda i: (i, 0)),
            pl.BlockSpec((_RBLK, CHA), lambda i: (i, 0)),
        ],
        out_shape=[
            jax.ShapeDtypeStruct((BATCH, CHA), jnp.int32),
            jax.ShapeDtypeStruct((BATCH, CHA), jnp.int32),
        ],
    )(ids_t)


_mesh = plsc.VectorSubcoreMesh(
    core_axis_name="c", subcore_axis_name="s", num_cores=NC, num_subcores=NS
)


def _make_sc_gather(half):
    @functools.partial(
        pl.kernel,
        out_type=jax.ShapeDtypeStruct((HB, SEQ, HIDDEN), jnp.float32),
        mesh=_mesh,
        scratch_types=[
            pltpu.VMEM((RPW, CHA), jnp.int32),
            pltpu.VMEM((RPW, CHA), jnp.int32),
            [pltpu.VMEM((CHA, HIDDEN), jnp.float32) for _ in range(NBUF)],
            [pltpu.VMEM((CHB, HIDDEN), jnp.float32) for _ in range(NBUF)],
            [pltpu.SemaphoreType.DMA for _ in range(NBUF)],
            [pltpu.SemaphoreType.DMA for _ in range(NBUF)],
            [pltpu.SemaphoreType.DMA for _ in range(NBUF)],
            [pltpu.SemaphoreType.DMA for _ in range(NBUF)],
        ],
        compiler_params=pltpu.CompilerParams(use_tc_tiling_on_sc=False),
    )
    def _sc_gather(table_hbm, idxa_hbm, idxb_hbm, out_hbm,
                   idxa_v, idxb_v, bufA, bufB, gsA, gsB, osA, osB):
        wid = lax.axis_index("s") * NC + lax.axis_index("c")
        row0 = wid * RPW
        pltpu.sync_copy(idxa_hbm.at[pl.ds(half * HB + row0, RPW)], idxa_v)
        pltpu.sync_copy(idxb_hbm.at[pl.ds(half * HB + row0, RPW)], idxb_v)

        def start_gather(r, b):
            pltpu.async_copy(table_hbm.at[idxa_v.at[r]], bufA[b], gsA[b])
            pltpu.async_copy(
                table_hbm.at[idxb_v.at[r, pl.ds(0, CHB)]], bufB[b], gsB[b]
            )

        def wait_gather(b):
            pltpu.make_async_copy(table_hbm.at[idxa_v.at[0]], bufA[b], gsA[b]).wait()
            pltpu.make_async_copy(
                table_hbm.at[idxb_v.at[0, pl.ds(0, CHB)]], bufB[b], gsB[b]
            ).wait()

        def start_out(r, b):
            row = row0 + r
            pltpu.async_copy(bufA[b], out_hbm.at[row, pl.ds(0, CHA)], osA[b])
            pltpu.async_copy(bufB[b], out_hbm.at[row, pl.ds(CHA, CHB)], osB[b])

        def wait_out(b):
            pltpu.make_async_copy(
                bufA[b], out_hbm.at[0, pl.ds(0, CHA)], osA[b]
            ).wait()
            pltpu.make_async_copy(
                bufB[b], out_hbm.at[0, pl.ds(CHA, CHB)], osB[b]
            ).wait()

        for b in range(NBUF):
            start_gather(b, b)

        def group(g, carry):
            for b in range(NBUF):
                wait_gather(b)
                start_out(g * NBUF + b, b)
            for b in range(NBUF):
                nr = (g + 1) * NBUF + b

                @pl.when(nr < RPW)
                def _():
                    wait_out(b)
                    start_gather(nr, b)

            return carry

        lax.fori_loop(0, NGROUP, group, 0)
        for b in range(NBUF):
            wait_out(b)

    return _sc_gather


_sc_gather_half = [_make_sc_gather(h) for h in range(NHALF)]


_TB = 128
_TSTEPS = HB // _TB


def _make_out_transpose(half):
    def _otr_body(*refs):
        if half == 0:
            x_hbm, out_ref, buf, sems = refs
        else:
            _, x_hbm, out_ref, buf, sems = refs
        g = pl.program_id(0)
        slot = lax.rem(g, 2)

        def start(step, sl):
            pltpu.make_async_copy(
                x_hbm.at[pl.ds(step * _TB, _TB)], buf.at[sl], sems.at[sl]
            ).start()

        @pl.when(g == 0)
        def _():
            start(g, slot)

        @pl.when(g + 1 < _TSTEPS)
        def _():
            start(g + 1, 1 - slot)

        pltpu.make_async_copy(
            x_hbm.at[pl.ds(0, _TB)], buf.at[slot], sems.at[slot]
        ).wait()
        w = buf[slot]
        for k in range(SEQ // 2):
            r = jnp.transpose(w[:, 2 * k : 2 * k + 2, :].reshape(_TB, 128))
            out_ref[pl.ds(2 * k, 2)] = r.reshape(2, HIDDEN, _TB)

    nin = 1 if half == 0 else 2
    return pl.pallas_call(
        _otr_body,
        grid=(_TSTEPS,),
        in_specs=[pl.BlockSpec(memory_space=pl.ANY)] * nin,
        out_specs=pl.BlockSpec(
            (SEQ, HIDDEN, _TB), lambda g: (0, 0, g + half * _TSTEPS)
        ),
        out_shape=jax.ShapeDtypeStruct((SEQ, HIDDEN, BATCH), jnp.float32),
        scratch_shapes=[
            pltpu.VMEM((2, _TB, SEQ, HIDDEN), jnp.float32),
            pltpu.SemaphoreType.DMA((2,)),
        ],
        input_output_aliases={} if half == 0 else {0: 0},
    )


_otr_half = [_make_out_transpose(h) for h in range(NHALF)]


def kernel(input_ids, base_table, lora_A, lora_B):
    tab = _fused_table(base_table.T, lora_A.T, lora_B)
    ua, ub = _remap_ids(input_ids.astype(jnp.int32).T)
    tab64 = tab.reshape(2 * TROWS, HIDDEN)
    x0 = _sc_gather_half[0](tab64, ua, ub)
    x1 = _sc_gather_half[1](tab64, ua, ub)
    p = _otr_half[0](x0)
    p = _otr_half[1](p, x1)
    return jnp.transpose(p, (2, 0, 1))
